# Initial kernel scaffold; baseline (speedup 1.0000x reference)
import jax
import jax.numpy as jnp
from jax.experimental import pallas as pl


def kernel(X, Z, W1, b1, W2, b2, W3, b3):
    raise NotImplementedError("write your pallas kernel here")


# speedup vs baseline: 2.1129x; 2.1129x over previous
import jax
import jax.numpy as jnp
from jax.experimental import pallas as pl

B, N, D = 16, 512, 128
E = 8
H1, H2 = 256, 256


def _dense_body(z_ref, x_ref, w1_ref, b1_ref, w2_ref, b2_ref, w3_ref, b3_ref,
                out_ref):
    x = x_ref[0]
    z = z_ref[0]
    acc = jnp.zeros((N, 1), jnp.float32)
    for e in range(E):
        h = jnp.tanh(jnp.dot(x, w1_ref[e]) + b1_ref[e:e + 1, :])
        h = jnp.tanh(jnp.dot(h, w2_ref[e]) + b2_ref[e:e + 1, :])
        y = jnp.sum(h * w3_ref[e:e + 1, :], axis=1, keepdims=True)
        y = y + b3_ref[e:e + 1, 0:1]
        mask = (z == e).astype(jnp.float32)
        acc = acc + y * mask
    s = jnp.sum(acc)
    out_ref[...] = jnp.broadcast_to(s, (1, 1, 128))


def kernel(X, Z, W1, b1, W2, b2, W3, b3):
    Zc = Z.reshape(B, N, 1).astype(jnp.int32)
    w3s = W3.reshape(E, H2)
    b3b = jnp.broadcast_to(b3.reshape(E, 1), (E, 128))
    out = pl.pallas_call(
        _dense_body,
        grid=(B,),
        in_specs=[
            pl.BlockSpec((1, N, 1), lambda b: (b, 0, 0)),
            pl.BlockSpec((1, N, D), lambda b: (b, 0, 0)),
            pl.BlockSpec((E, D, H1), lambda b: (0, 0, 0)),
            pl.BlockSpec((E, H1), lambda b: (0, 0)),
            pl.BlockSpec((E, H1, H2), lambda b: (0, 0, 0)),
            pl.BlockSpec((E, H2), lambda b: (0, 0)),
            pl.BlockSpec((E, H2), lambda b: (0, 0)),
            pl.BlockSpec((E, 128), lambda b: (0, 0)),
        ],
        out_specs=pl.BlockSpec((1, 1, 128), lambda b: (b, 0, 0)),
        out_shape=jax.ShapeDtypeStruct((B, 1, 128), jnp.float32),
    )(Zc, X, W1, b1, W2, b2, w3s, b3b)
    return out[:, 0, 0]
